# baseline (speedup 1.0000x reference)
import functools
import jax
import jax.numpy as jnp
from jax import lax
from jax.experimental import pallas as pl
from jax.experimental.pallas import tpu as pltpu
from jax.experimental.pallas import tpu_sc as plsc

N = 50000
E = 800000
D = 64
DE = 16
EPS = 1e-5

BE = 3200
BN_BLK = 2000


def _k1_body(nbr_ref, w1_ref, b1_ref, w2_ref, b2_ref, w3_ref, b3_ref,
             t0_ref, t1_ref):
    x = nbr_ref[...]
    h = jnp.maximum(jnp.dot(x, w1_ref[...], preferred_element_type=jnp.float32)
                    + b1_ref[...], 0.0)
    h = jnp.maximum(jnp.dot(h, w2_ref[...], preferred_element_type=jnp.float32)
                    + b2_ref[...], 0.0)
    t = jnp.dot(h, w3_ref[...], preferred_element_type=jnp.float32) + b3_ref[...]
    t0_ref[...] = t[:, :32]
    t1_ref[...] = t[:, 32:]


def _edge_mlp(nbr_fea, W1, b1, W2, b2, W3, b3):
    grid = (E // BE,)
    full = lambda shape: pl.BlockSpec(shape, lambda i: (0, 0))
    return pl.pallas_call(
        _k1_body,
        grid=grid,
        in_specs=[
            pl.BlockSpec((BE, DE), lambda i: (i, 0)),
            full((DE, 256)), full((1, 256)),
            full((256, 128)), full((1, 128)),
            full((128, D)), full((1, D)),
        ],
        out_specs=[
            pl.BlockSpec((BE, 32), lambda i: (i, 0)),
            pl.BlockSpec((BE, 32), lambda i: (i, 0)),
        ],
        out_shape=[
            jax.ShapeDtypeStruct((E, 32), jnp.float32),
            jax.ShapeDtypeStruct((E, 32), jnp.float32),
        ],
    )(nbr_fea, W1, b1.reshape(1, 256), W2, b2.reshape(1, 128),
      W3, b3.reshape(1, D))


def _stats_body(x_ref, o_ref):
    i = pl.program_id(0)
    blk = x_ref[...]
    s = jnp.sum(blk, axis=0, keepdims=True)
    ss = jnp.sum(blk * blk, axis=0, keepdims=True)
    val = jnp.concatenate([s, ss], axis=0)

    @pl.when(i == 0)
    def _():
        o_ref[...] = val

    @pl.when(i > 0)
    def _():
        o_ref[...] = o_ref[...] + val


def _bn_stats(x):
    grid = (N // BN_BLK,)
    return pl.pallas_call(
        _stats_body,
        grid=grid,
        in_specs=[pl.BlockSpec((BN_BLK, D), lambda i: (i, 0))],
        out_specs=pl.BlockSpec((2, D), lambda i: (0, 0)),
        out_shape=jax.ShapeDtypeStruct((2, D), jnp.float32),
    )(x)


def _combine_body(a_ref, st_ref, gi_ref, bi_ref, acc_ref, anorm_ref, comb_ref):
    st = st_ref[...]
    m = st[0:1, :] / N
    v = st[1:2, :] / N - m * m
    scale = gi_ref[...] * lax.rsqrt(v + EPS)
    a_norm = (a_ref[...] - m) * scale + bi_ref[...]
    acc = acc_ref[...]
    s = acc[0] + acc[1]
    add = jnp.concatenate([s[0], s[1]], axis=-1)
    anorm_ref[...] = a_norm
    comb_ref[...] = a_norm + add


def _combine(atom_fea, stats, gi, bi, acc):
    grid = (N // BN_BLK,)
    return pl.pallas_call(
        _combine_body,
        grid=grid,
        in_specs=[
            pl.BlockSpec((BN_BLK, D), lambda i: (i, 0)),
            pl.BlockSpec((2, D), lambda i: (0, 0)),
            pl.BlockSpec((1, D), lambda i: (0, 0)),
            pl.BlockSpec((1, D), lambda i: (0, 0)),
            pl.BlockSpec((2, 2, BN_BLK, 32), lambda i: (0, 0, i, 0)),
        ],
        out_specs=[
            pl.BlockSpec((BN_BLK, D), lambda i: (i, 0)),
            pl.BlockSpec((BN_BLK, D), lambda i: (i, 0)),
        ],
        out_shape=[
            jax.ShapeDtypeStruct((N, D), jnp.float32),
            jax.ShapeDtypeStruct((N, D), jnp.float32),
        ],
    )(atom_fea, stats, gi.reshape(1, D), bi.reshape(1, D), acc)


def _k4_body(xi_ref, xj_ref, wa_ref, wb_ref, b4_ref, w5_ref, b5_ref,
             h0_ref, h1_ref):
    xi = xi_ref[...]
    xj = xj_ref[...]
    g = (jnp.dot(xi, wa_ref[...], preferred_element_type=jnp.float32)
         + jnp.dot(xj, wb_ref[...], preferred_element_type=jnp.float32)
         + b4_ref[...])
    g = jnp.maximum(g, 0.0)
    h = jnp.dot(g, w5_ref[...], preferred_element_type=jnp.float32) + b5_ref[...]
    h0_ref[...] = h[:, :32]
    h1_ref[...] = h[:, 32:]


def _edgeconv_mlp(x_i, x_j, W4, b4, W5, b5):
    wa = W4[:D, :] - W4[D:, :]
    wb = W4[D:, :]
    grid = (E // BE,)
    full = lambda shape: pl.BlockSpec(shape, lambda i: (0, 0))
    return pl.pallas_call(
        _k4_body,
        grid=grid,
        in_specs=[
            pl.BlockSpec((BE, D), lambda i: (i, 0)),
            pl.BlockSpec((BE, D), lambda i: (i, 0)),
            full((D, 256)), full((D, 256)), full((1, 256)),
            full((256, D)), full((1, D)),
        ],
        out_specs=[
            pl.BlockSpec((BE, 32), lambda i: (i, 0)),
            pl.BlockSpec((BE, 32), lambda i: (i, 0)),
        ],
        out_shape=[
            jax.ShapeDtypeStruct((E, 32), jnp.float32),
            jax.ShapeDtypeStruct((E, 32), jnp.float32),
        ],
    )(x_i, x_j, wa, wb, b4.reshape(1, 256), W5, b5.reshape(1, D))


def _mean_stats_body(acc_ref, cnt_ref, y_ref, st_ref):
    i = pl.program_id(0)
    acc = acc_ref[...]
    s = acc[0] + acc[1]
    summed = jnp.concatenate([s[0], s[1]], axis=-1)
    cnt = cnt_ref[...]
    c = jnp.maximum(cnt[0] + cnt[1], 1.0)
    y = summed / c
    y_ref[...] = y
    sm = jnp.sum(y, axis=0, keepdims=True)
    ss = jnp.sum(y * y, axis=0, keepdims=True)
    val = jnp.concatenate([sm, ss], axis=0)

    @pl.when(i == 0)
    def _():
        st_ref[...] = val

    @pl.when(i > 0)
    def _():
        st_ref[...] = st_ref[...] + val


def _mean_and_stats(acc, cnt):
    grid = (N // BN_BLK,)
    return pl.pallas_call(
        _mean_stats_body,
        grid=grid,
        in_specs=[
            pl.BlockSpec((2, 2, BN_BLK, 32), lambda i: (0, 0, i, 0)),
            pl.BlockSpec((2, BN_BLK, 1), lambda i: (0, i, 0)),
        ],
        out_specs=[
            pl.BlockSpec((BN_BLK, D), lambda i: (i, 0)),
            pl.BlockSpec((2, D), lambda i: (0, 0)),
        ],
        out_shape=[
            jax.ShapeDtypeStruct((N, D), jnp.float32),
            jax.ShapeDtypeStruct((2, D), jnp.float32),
        ],
    )(acc, cnt)


def _final_body(y_ref, st_ref, go_ref, bo_ref, comb_ref, out_ref):
    st = st_ref[...]
    m = st[0:1, :] / N
    v = st[1:2, :] / N - m * m
    scale = go_ref[...] * lax.rsqrt(v + EPS)
    out_ref[...] = (y_ref[...] - m) * scale + bo_ref[...] + comb_ref[...]


def _final(y, stats, go, bo, combined):
    grid = (N // BN_BLK,)
    return pl.pallas_call(
        _final_body,
        grid=grid,
        in_specs=[
            pl.BlockSpec((BN_BLK, D), lambda i: (i, 0)),
            pl.BlockSpec((2, D), lambda i: (0, 0)),
            pl.BlockSpec((1, D), lambda i: (0, 0)),
            pl.BlockSpec((1, D), lambda i: (0, 0)),
            pl.BlockSpec((BN_BLK, D), lambda i: (i, 0)),
        ],
        out_specs=pl.BlockSpec((BN_BLK, D), lambda i: (i, 0)),
        out_shape=jax.ShapeDtypeStruct((N, D), jnp.float32),
    )(y, stats, go.reshape(1, D), bo.reshape(1, D), combined)


def _scatter_t(t0, t1, src, tgt):
    z = jnp.zeros((N, 32), jnp.float32)
    half = E // 2
    acc = []
    for c in range(2):
        sl = slice(c * half, (c + 1) * half)
        a0 = z.at[src[sl]].add(t0[sl]).at[tgt[sl]].add(t0[sl])
        a1 = z.at[src[sl]].add(t1[sl]).at[tgt[sl]].add(t1[sl])
        acc.append(jnp.stack([a0, a1]))
    cnt = []
    for c in range(2):
        sl = slice(c * half, (c + 1) * half)
        cnt.append(jnp.zeros((N,), jnp.float32).at[tgt[sl]].add(1.0))
    return jnp.stack(acc), jnp.stack(cnt)[:, :, None]


def _gather_rows(a_norm, combined, src, tgt):
    return a_norm[tgt], combined[src]


def _scatter_h(h0, h1, tgt):
    z = jnp.zeros((N, 32), jnp.float32)
    half = E // 2
    acc = []
    for c in range(2):
        sl = slice(c * half, (c + 1) * half)
        a0 = z.at[tgt[sl]].add(h0[sl])
        a1 = z.at[tgt[sl]].add(h1[sl])
        acc.append(jnp.stack([a0, a1]))
    return jnp.stack(acc)


def kernel(atom_fea, nbr_fea, edge_index, W1, b1, W2, b2, W3, b3, W4, b4,
           W5, b5, gi, bi, go, bo):
    src = edge_index[0]
    tgt = edge_index[1]

    t0, t1 = _edge_mlp(nbr_fea, W1, b1, W2, b2, W3, b3)
    stats_a = _bn_stats(atom_fea)
    acc_t, cnt = _scatter_t(t0, t1, src, tgt)
    a_norm, combined = _combine(atom_fea, stats_a, gi, bi, acc_t)
    x_i, x_j = _gather_rows(a_norm, combined, src, tgt)
    h0, h1 = _edgeconv_mlp(x_i, x_j, W4, b4, W5, b5)
    acc_h = _scatter_h(h0, h1, tgt)
    y, stats_y = _mean_and_stats(acc_h, cnt)
    return _final(y, stats_y, go, bo, combined)
